# initial kernel scaffold (unmeasured)
import jax
import jax.numpy as jnp
from jax import lax
from jax.experimental import pallas as pl
from jax.experimental.pallas import tpu as pltpu

N_DEV = 4
EPS = 1e-5


def kernel(x, t_emb, W_scale, W_shift):
    b, s, c_local = x.shape
    c_global = c_local * N_DEV

    def body(x_ref, t_ref, ws_ref, wsh_ref, out_ref,
             local_ref, rstats_ref, send_sems, recv_sems):
        my = lax.axis_index("i")

        barrier = pltpu.get_barrier_semaphore()
        for d in (1, 2, 3):
            pl.semaphore_signal(
                barrier, inc=1,
                device_id=((my + d) % N_DEV,),
                device_id_type=pl.DeviceIdType.MESH,
            )
        pl.semaphore_wait(barrier, N_DEV - 1)

        xs = x_ref[...]
        local_ref[0:b, :] = jnp.sum(xs, axis=-1)
        local_ref[b:2 * b, :] = jnp.sum(xs * xs, axis=-1)

        rdmas = []
        for d in (1, 2, 3):
            slot = (N_DEV - d) - 1
            r = pltpu.make_async_remote_copy(
                src_ref=local_ref,
                dst_ref=rstats_ref.at[slot],
                send_sem=send_sems.at[d - 1],
                recv_sem=recv_sems.at[slot],
                device_id=((my + d) % N_DEV,),
                device_id_type=pl.DeviceIdType.MESH,
            )
            r.start()
            rdmas.append(r)

        scale = jnp.dot(t_ref[...], ws_ref[...],
                        preferred_element_type=jnp.float32)
        shift = jnp.dot(t_ref[...], wsh_ref[...],
                        preferred_element_type=jnp.float32)

        for r in rdmas:
            r.wait()

        tot = local_ref[...] + rstats_ref[0] + rstats_ref[1] + rstats_ref[2]
        mean = tot[0:b, :] * (1.0 / c_global)
        ex2 = tot[b:2 * b, :] * (1.0 / c_global)
        inv = lax.rsqrt(ex2 - mean * mean + EPS)

        h = (xs - mean[:, :, None]) * inv[:, :, None]
        out_ref[...] = h * (1.0 + scale[:, None, :]) + shift[:, None, :]

    return pl.pallas_call(
        body,
        out_shape=jax.ShapeDtypeStruct(x.shape, jnp.float32),
        in_specs=[pl.BlockSpec(memory_space=pltpu.VMEM)] * 4,
        out_specs=pl.BlockSpec(memory_space=pltpu.VMEM),
        scratch_shapes=[
            pltpu.VMEM((2 * b, s), jnp.float32),
            pltpu.VMEM((N_DEV - 1, 2 * b, s), jnp.float32),
            pltpu.SemaphoreType.DMA((N_DEV - 1,)),
            pltpu.SemaphoreType.DMA((N_DEV - 1,)),
        ],
        compiler_params=pltpu.CompilerParams(collective_id=0),
    )(x, t_emb, W_scale, W_shift)


# baseline (device time: 38976 ns/iter reference)
import jax
import jax.numpy as jnp
from jax import lax
from jax.experimental import pallas as pl
from jax.experimental.pallas import tpu as pltpu

N_DEV = 4
EPS = 1e-5


def kernel(x, t_emb, W_scale, W_shift):
    b, s, c_local = x.shape
    c_global = c_local * N_DEV

    def body(x_ref, t_ref, ws_ref, wsh_ref, out_ref,
             local_ref, rstats_ref, send_sems, recv_sems):
        my = lax.axis_index("i")

        barrier = pltpu.get_barrier_semaphore()
        for d in (1, 2, 3):
            pl.semaphore_signal(
                barrier, inc=1,
                device_id=((my + d) % N_DEV,),
                device_id_type=pl.DeviceIdType.MESH,
            )
        pl.semaphore_wait(barrier, N_DEV - 1)

        xs = x_ref[...]
        local_ref[0:b, :] = jnp.sum(xs, axis=-1)
        local_ref[b:2 * b, :] = jnp.sum(xs * xs, axis=-1)

        rdmas = []
        for d in (1, 2, 3):
            slot = (N_DEV - d) - 1
            r = pltpu.make_async_remote_copy(
                src_ref=local_ref,
                dst_ref=rstats_ref.at[slot],
                send_sem=send_sems.at[d - 1],
                recv_sem=recv_sems.at[slot],
                device_id=((my + d) % N_DEV,),
                device_id_type=pl.DeviceIdType.MESH,
            )
            r.start()
            rdmas.append(r)

        scale = jnp.dot(t_ref[...], ws_ref[...],
                        preferred_element_type=jnp.float32)
        shift = jnp.dot(t_ref[...], wsh_ref[...],
                        preferred_element_type=jnp.float32)

        for r in rdmas:
            r.wait()

        tot = local_ref[...] + rstats_ref[0] + rstats_ref[1] + rstats_ref[2]
        mean = tot[0:b, :] * (1.0 / c_global)
        ex2 = tot[b:2 * b, :] * (1.0 / c_global)
        inv = lax.rsqrt(ex2 - mean * mean + EPS)

        h = (xs - mean[:, :, None]) * inv[:, :, None]
        out_ref[...] = h * (1.0 + scale[:, None, :]) + shift[:, None, :]

    return pl.pallas_call(
        body,
        out_shape=jax.ShapeDtypeStruct(x.shape, jnp.float32),
        in_specs=[pl.BlockSpec(memory_space=pltpu.VMEM)] * 4,
        out_specs=pl.BlockSpec(memory_space=pltpu.VMEM),
        scratch_shapes=[
            pltpu.VMEM((2 * b, s), jnp.float32),
            pltpu.VMEM((N_DEV - 1, 2 * b, s), jnp.float32),
            pltpu.SemaphoreType.DMA((N_DEV - 1,)),
            pltpu.SemaphoreType.DMA((N_DEV - 1,)),
        ],
        compiler_params=pltpu.CompilerParams(
            collective_id=0, vmem_limit_bytes=100 * 1024 * 1024
        ),
    )(x, t_emb, W_scale, W_shift)


# device time: 34763 ns/iter; 1.1212x vs baseline; 1.1212x over previous
import jax
import jax.numpy as jnp
from jax import lax
from jax.experimental import pallas as pl
from jax.experimental.pallas import tpu as pltpu

N_DEV = 4
EPS = 1e-5
BS = 512
LAG = 2


def kernel(x, t_emb, W_scale, W_shift):
    b, s, c_local = x.shape
    c_global = c_local * N_DEV
    nb = s // BS
    n_steps = nb + LAG

    def body(x_ref, t_ref, ws_ref, wsh_ref, out_ref,
             xsave_ref, lstats_ref, rstats_ref, send_sems, recv_sems):
        k = pl.program_id(0)
        my = lax.axis_index("i")

        @pl.when(k == 0)
        def _entry_barrier():
            barrier = pltpu.get_barrier_semaphore()
            for d in (1, 2, 3):
                pl.semaphore_signal(
                    barrier, inc=1,
                    device_id=((my + d) % N_DEV,),
                    device_id_type=pl.DeviceIdType.MESH,
                )
            pl.semaphore_wait(barrier, N_DEV - 1)

        @pl.when(k < nb)
        def _phase_a():
            xs = x_ref[...]
            lstats_ref[k, 0:b, :] = jnp.sum(xs, axis=-1)
            lstats_ref[k, b:2 * b, :] = jnp.sum(xs * xs, axis=-1)
            xsave_ref[lax.rem(k, LAG + 1)] = xs
            for d in (1, 2, 3):
                slot = (N_DEV - d) - 1
                r = pltpu.make_async_remote_copy(
                    src_ref=lstats_ref.at[k],
                    dst_ref=rstats_ref.at[k, slot],
                    send_sem=send_sems.at[k, d - 1],
                    recv_sem=recv_sems.at[k, slot],
                    device_id=((my + d) % N_DEV,),
                    device_id_type=pl.DeviceIdType.MESH,
                )
                r.start()

        @pl.when(k >= LAG)
        def _phase_b():
            j = k - LAG
            for i in (0, 1, 2):
                r = pltpu.make_async_remote_copy(
                    src_ref=lstats_ref.at[j],
                    dst_ref=rstats_ref.at[j, i],
                    send_sem=send_sems.at[j, i],
                    recv_sem=recv_sems.at[j, i],
                    device_id=(my,),
                    device_id_type=pl.DeviceIdType.MESH,
                )
                r.wait_recv()
                r.wait_send()

            tot = (lstats_ref[j] + rstats_ref[j, 0]
                   + rstats_ref[j, 1] + rstats_ref[j, 2])
            mean = tot[0:b, :] * (1.0 / c_global)
            ex2 = tot[b:2 * b, :] * (1.0 / c_global)
            inv = lax.rsqrt(ex2 - mean * mean + EPS)

            scale = jnp.dot(t_ref[...], ws_ref[...],
                            preferred_element_type=jnp.float32)
            shift = jnp.dot(t_ref[...], wsh_ref[...],
                            preferred_element_type=jnp.float32)

            xj = xsave_ref[lax.rem(j, LAG + 1)]
            h = (xj - mean[:, :, None]) * inv[:, :, None]
            out_ref[...] = h * (1.0 + scale[:, None, :]) + shift[:, None, :]

    grid = (n_steps,)
    return pl.pallas_call(
        body,
        grid=grid,
        out_shape=jax.ShapeDtypeStruct(x.shape, jnp.float32),
        in_specs=[
            pl.BlockSpec((b, BS, c_local),
                         lambda k: (0, jnp.minimum(k, nb - 1), 0),
                         memory_space=pltpu.VMEM),
            pl.BlockSpec((b, t_emb.shape[1]), lambda k: (0, 0),
                         memory_space=pltpu.VMEM),
            pl.BlockSpec(W_scale.shape, lambda k: (0, 0),
                         memory_space=pltpu.VMEM),
            pl.BlockSpec(W_shift.shape, lambda k: (0, 0),
                         memory_space=pltpu.VMEM),
        ],
        out_specs=pl.BlockSpec((b, BS, c_local),
                               lambda k: (0, jnp.maximum(k - LAG, 0), 0),
                               memory_space=pltpu.VMEM),
        scratch_shapes=[
            pltpu.VMEM((LAG + 1, b, BS, c_local), jnp.float32),
            pltpu.VMEM((nb, 2 * b, BS), jnp.float32),
            pltpu.VMEM((nb, N_DEV - 1, 2 * b, BS), jnp.float32),
            pltpu.SemaphoreType.DMA((nb, N_DEV - 1)),
            pltpu.SemaphoreType.DMA((nb, N_DEV - 1)),
        ],
        compiler_params=pltpu.CompilerParams(
            collective_id=0,
            vmem_limit_bytes=100 * 1024 * 1024,
            dimension_semantics=("arbitrary",),
        ),
    )(x, t_emb, W_scale, W_shift)


# device time: 25909 ns/iter; 1.5043x vs baseline; 1.3417x over previous
import jax
import jax.numpy as jnp
from jax import lax
from jax.experimental import pallas as pl
from jax.experimental.pallas import tpu as pltpu

N_DEV = 4
EPS = 1e-5
BS = 512
LAG = 2


def kernel(x, t_emb, W_scale, W_shift):
    b, s, c_local = x.shape
    c_global = c_local * N_DEV
    nb = s // BS
    n_steps = nb + LAG

    def body(x_ref, t_ref, ws_ref, wsh_ref, out_ref,
             xsave_ref, lstats_ref, rstats_ref, send_sems, recv_sems):
        k = pl.program_id(0)
        my = lax.axis_index("i")

        barrier = pltpu.get_barrier_semaphore()

        @pl.when(k == 0)
        def _entry_barrier_signal():
            for d in (1, 2, 3):
                pl.semaphore_signal(
                    barrier, inc=1,
                    device_id=((my + d) % N_DEV,),
                    device_id_type=pl.DeviceIdType.MESH,
                )

        @pl.when(k < nb)
        def _phase_a_stats():
            xs = x_ref[...]
            lstats_ref[k] = jnp.concatenate(
                [jnp.sum(xs, axis=-1), jnp.sum(xs * xs, axis=-1)], axis=0)
            xsave_ref[lax.rem(k, LAG + 1)] = xs

        @pl.when(k == 0)
        def _entry_barrier_wait():
            pl.semaphore_wait(barrier, N_DEV - 1)

        @pl.when(k < nb)
        def _phase_a_send():
            for d in (1, 2, 3):
                slot = (N_DEV - d) - 1
                r = pltpu.make_async_remote_copy(
                    src_ref=lstats_ref.at[k],
                    dst_ref=rstats_ref.at[k, slot],
                    send_sem=send_sems.at[k, d - 1],
                    recv_sem=recv_sems.at[k, slot],
                    device_id=((my + d) % N_DEV,),
                    device_id_type=pl.DeviceIdType.MESH,
                )
                r.start()

        @pl.when(k >= LAG)
        def _phase_b():
            j = k - LAG
            for i in (0, 1, 2):
                r = pltpu.make_async_remote_copy(
                    src_ref=lstats_ref.at[j],
                    dst_ref=rstats_ref.at[j, i],
                    send_sem=send_sems.at[j, i],
                    recv_sem=recv_sems.at[j, i],
                    device_id=(my,),
                    device_id_type=pl.DeviceIdType.MESH,
                )
                r.wait_recv()
                r.wait_send()

            tot = (lstats_ref[j] + rstats_ref[j, 0]
                   + rstats_ref[j, 1] + rstats_ref[j, 2])
            mean = tot[0:b, :] * (1.0 / c_global)
            ex2 = tot[b:2 * b, :] * (1.0 / c_global)
            inv = lax.rsqrt(ex2 - mean * mean + EPS)

            scale = jnp.dot(t_ref[...], ws_ref[...],
                            preferred_element_type=jnp.float32)
            shift = jnp.dot(t_ref[...], wsh_ref[...],
                            preferred_element_type=jnp.float32)

            xj = xsave_ref[lax.rem(j, LAG + 1)]
            h = (xj - mean[:, :, None]) * inv[:, :, None]
            out_ref[...] = h * (1.0 + scale[:, None, :]) + shift[:, None, :]

    grid = (n_steps,)
    return pl.pallas_call(
        body,
        grid=grid,
        out_shape=jax.ShapeDtypeStruct(x.shape, jnp.float32),
        in_specs=[
            pl.BlockSpec((b, BS, c_local),
                         lambda k: (0, jnp.minimum(k, nb - 1), 0),
                         memory_space=pltpu.VMEM),
            pl.BlockSpec((b, t_emb.shape[1]), lambda k: (0, 0),
                         memory_space=pltpu.VMEM),
            pl.BlockSpec(W_scale.shape, lambda k: (0, 0),
                         memory_space=pltpu.VMEM),
            pl.BlockSpec(W_shift.shape, lambda k: (0, 0),
                         memory_space=pltpu.VMEM),
        ],
        out_specs=pl.BlockSpec((b, BS, c_local),
                               lambda k: (0, jnp.maximum(k - LAG, 0), 0),
                               memory_space=pltpu.VMEM),
        scratch_shapes=[
            pltpu.VMEM((LAG + 1, b, BS, c_local), jnp.float32),
            pltpu.VMEM((nb, 2 * b, BS), jnp.float32),
            pltpu.VMEM((nb, N_DEV - 1, 2 * b, BS), jnp.float32),
            pltpu.SemaphoreType.DMA((nb, N_DEV - 1)),
            pltpu.SemaphoreType.DMA((nb, N_DEV - 1)),
        ],
        compiler_params=pltpu.CompilerParams(collective_id=0),
    )(x, t_emb, W_scale, W_shift)


# device time: 25296 ns/iter; 1.5408x vs baseline; 1.0242x over previous
import jax
import jax.numpy as jnp
from jax import lax
from jax.experimental import pallas as pl
from jax.experimental.pallas import tpu as pltpu

N_DEV = 4
EPS = 1e-5
BS = 512
LAG = 3


def kernel(x, t_emb, W_scale, W_shift):
    b, s, c_local = x.shape
    c_global = c_local * N_DEV
    nb = s // BS
    n_steps = nb + LAG

    def body(x_ref, t_ref, ws_ref, wsh_ref, out_ref,
             xsave_ref, lstats_ref, rstats_ref, mod_ref,
             send_sems, recv_sems):
        k = pl.program_id(0)
        my = lax.axis_index("i")

        barrier = pltpu.get_barrier_semaphore()

        @pl.when(k == 0)
        def _entry_barrier_signal():
            for d in (1, 2, 3):
                pl.semaphore_signal(
                    barrier, inc=1,
                    device_id=((my + d) % N_DEV,),
                    device_id_type=pl.DeviceIdType.MESH,
                )

        @pl.when(k == 0)
        def _modulation():
            mod_ref[0] = 1.0 + jnp.dot(t_ref[...], ws_ref[...],
                                       preferred_element_type=jnp.float32)
            mod_ref[1] = jnp.dot(t_ref[...], wsh_ref[...],
                                 preferred_element_type=jnp.float32)

        @pl.when(k < nb)
        def _phase_a_stats():
            xs = x_ref[...]
            lstats_ref[k] = jnp.concatenate(
                [jnp.sum(xs, axis=-1), jnp.sum(xs * xs, axis=-1)], axis=0)
            xsave_ref[lax.rem(k, LAG + 1)] = xs

        @pl.when(k == 0)
        def _entry_barrier_wait():
            pl.semaphore_wait(barrier, N_DEV - 1)

        @pl.when(k < nb)
        def _phase_a_send():
            for d in (1, 2, 3):
                slot = (N_DEV - d) - 1
                r = pltpu.make_async_remote_copy(
                    src_ref=lstats_ref.at[k],
                    dst_ref=rstats_ref.at[k, slot],
                    send_sem=send_sems.at[k, d - 1],
                    recv_sem=recv_sems.at[k, slot],
                    device_id=((my + d) % N_DEV,),
                    device_id_type=pl.DeviceIdType.MESH,
                )
                r.start()

        @pl.when(k >= LAG)
        def _phase_b():
            j = k - LAG
            for i in (0, 1, 2):
                r = pltpu.make_async_remote_copy(
                    src_ref=lstats_ref.at[j],
                    dst_ref=rstats_ref.at[j, i],
                    send_sem=send_sems.at[j, i],
                    recv_sem=recv_sems.at[j, i],
                    device_id=(my,),
                    device_id_type=pl.DeviceIdType.MESH,
                )
                r.wait_recv()
                r.wait_send()

            tot = (lstats_ref[j] + rstats_ref[j, 0]
                   + rstats_ref[j, 1] + rstats_ref[j, 2])
            mean = tot[0:b, :] * (1.0 / c_global)
            ex2 = tot[b:2 * b, :] * (1.0 / c_global)
            inv = lax.rsqrt(ex2 - mean * mean + EPS)

            xj = xsave_ref[lax.rem(j, LAG + 1)]
            h = (xj - mean[:, :, None]) * inv[:, :, None]
            out_ref[...] = (h * mod_ref[0][:, None, :]
                            + mod_ref[1][:, None, :])

    grid = (n_steps,)
    return pl.pallas_call(
        body,
        grid=grid,
        out_shape=jax.ShapeDtypeStruct(x.shape, jnp.float32),
        in_specs=[
            pl.BlockSpec((b, BS, c_local),
                         lambda k: (0, jnp.minimum(k, nb - 1), 0),
                         memory_space=pltpu.VMEM),
            pl.BlockSpec((b, t_emb.shape[1]), lambda k: (0, 0),
                         memory_space=pltpu.VMEM),
            pl.BlockSpec(W_scale.shape, lambda k: (0, 0),
                         memory_space=pltpu.VMEM),
            pl.BlockSpec(W_shift.shape, lambda k: (0, 0),
                         memory_space=pltpu.VMEM),
        ],
        out_specs=pl.BlockSpec((b, BS, c_local),
                               lambda k: (0, jnp.maximum(k - LAG, 0), 0),
                               memory_space=pltpu.VMEM),
        scratch_shapes=[
            pltpu.VMEM((LAG + 1, b, BS, c_local), jnp.float32),
            pltpu.VMEM((nb, 2 * b, BS), jnp.float32),
            pltpu.VMEM((nb, N_DEV - 1, 2 * b, BS), jnp.float32),
            pltpu.VMEM((2, b, c_local), jnp.float32),
            pltpu.SemaphoreType.DMA((nb, N_DEV - 1)),
            pltpu.SemaphoreType.DMA((nb, N_DEV - 1)),
        ],
        compiler_params=pltpu.CompilerParams(collective_id=0),
    )(x, t_emb, W_scale, W_shift)
